# baseline (speedup 1.0000x reference)
import functools

import jax
import jax.numpy as jnp
from jax import lax
from jax.experimental import pallas as pl
from jax.experimental.pallas import tpu as pltpu
from jax.experimental.pallas import tpu_sc as plsc

_B = 8
_N = 8192
_G = 512
_M = 32
_CB = 256


def _fps_body(xT_ref, idx_ref, cx_ref, cy_ref, cz_ref, dist_ref):
    x = xT_ref[:, 0, :]
    y = xT_ref[:, 1, :]
    z = xT_ref[:, 2, :]
    lane = lax.broadcasted_iota(jnp.int32, (_B, _N), 1)
    boff = lax.broadcasted_iota(jnp.int32, (_B, 1), 0) * _N
    glane = lax.broadcasted_iota(jnp.int32, (_B, _G), 1)
    dist_ref[...] = jnp.full((_B, _N), 1e10, jnp.float32)

    def step(i, carry):
        idx_a, cx_a, cy_a, cz_a = carry
        d = dist_ref[...]
        m = jnp.max(d, axis=1, keepdims=True)
        f = jnp.min(jnp.where(d == m, lane, _N), axis=1, keepdims=True)
        sel = lane == f
        cx = jnp.sum(jnp.where(sel, x, 0.0), axis=1, keepdims=True)
        cy = jnp.sum(jnp.where(sel, y, 0.0), axis=1, keepdims=True)
        cz = jnp.sum(jnp.where(sel, z, 0.0), axis=1, keepdims=True)
        hot = glane == i
        idx_a = jnp.where(hot, f + boff, idx_a)
        cx_a = jnp.where(hot, cx, cx_a)
        cy_a = jnp.where(hot, cy, cy_a)
        cz_a = jnp.where(hot, cz, cz_a)
        dx = x - cx
        dy = y - cy
        dz = z - cz
        dist_ref[...] = jnp.minimum(d, dx * dx + dy * dy + dz * dz)
        return (idx_a, cx_a, cy_a, cz_a)

    init = (jnp.zeros((_B, _G), jnp.int32),
            jnp.zeros((_B, _G), jnp.float32),
            jnp.zeros((_B, _G), jnp.float32),
            jnp.zeros((_B, _G), jnp.float32))
    idx_a, cx_a, cy_a, cz_a = lax.fori_loop(0, _G, step, init)
    idx_ref[...] = idx_a
    cx_ref[...] = cx_a
    cy_ref[...] = cy_a
    cz_ref[...] = cz_a


def _knn_body(xT_ref, cx_ref, cy_ref, cz_ref, idx_ref, d_ref):
    b = pl.program_id(0)
    x = xT_ref[:, 0, :]
    y = xT_ref[:, 1, :]
    z = xT_ref[:, 2, :]
    rr = lax.broadcasted_iota(jnp.int32, (_CB, _CB), 0)
    cc = lax.broadcasted_iota(jnp.int32, (_CB, _CB), 1)
    eye = rr == cc

    def tocol(row_ref):
        row = jnp.broadcast_to(row_ref[...].reshape(1, _CB), (_CB, _CB))
        return jnp.sum(jnp.where(eye, row, 0.0), axis=1, keepdims=True)

    cxc = tocol(cx_ref)
    cyc = tocol(cy_ref)
    czc = tocol(cz_ref)
    dx = cxc - x
    dy = cyc - y
    dz = czc - z
    d_ref[...] = dx * dx + dy * dy + dz * dz
    lane = lax.broadcasted_iota(jnp.int32, (_CB, _N), 1)
    klane = lax.broadcasted_iota(jnp.int32, (_CB, _M), 1)
    off = b * _N

    bigf = jnp.float32(3.4e38)

    _E = 16

    def step(kk, idx_a):
        cur = d_ref[...]
        for j in range(_E):
            m = jnp.min(cur, axis=1, keepdims=True)
            a = jnp.min(jnp.where(cur == m, lane, _N), axis=1, keepdims=True)
            idx_a = jnp.where(klane == _E * kk + j, a + off, idx_a)
            cur = jnp.where(lane == a, bigf, cur)
        d_ref[...] = cur
        return idx_a

    idx_a = lax.fori_loop(0, _M // _E, step, jnp.zeros((_CB, _M), jnp.int32))
    idx_ref[0, :, :] = idx_a


def _sc_gather_body(xyz_hbm, idx_hbm, cidx_hbm, out_hbm,
                    pts_v, idx_v, cidx_v, out_v):
    gpt = (_B * _G) // 32
    wid = lax.axis_index("s") * 2 + lax.axis_index("c")
    gbase = wid * gpt
    b = gbase // _G
    pbase = b * _N
    pltpu.sync_copy(xyz_hbm.at[pl.ds(pbase * 3, _N * 3)], pts_v)
    pltpu.sync_copy(idx_hbm.at[pl.ds(gbase * _M, gpt * _M)], idx_v)
    pltpu.sync_copy(cidx_hbm.at[pl.ds(gbase, gpt)], cidx_v)
    lane16 = lax.broadcasted_iota(jnp.int32, (16,), 0)

    def group(g, carry):
        gg = jnp.full((16,), g, jnp.int32)
        ci = plsc.load_gather(cidx_v, [gg])
        ca = (ci - pbase) * 3
        cxv = plsc.load_gather(pts_v, [ca])
        cyv = plsc.load_gather(pts_v, [ca + 1])
        czv = plsc.load_gather(pts_v, [ca + 2])
        for j in range(_M // 16):
            ii = idx_v[pl.ds(g * _M + j * 16, 16)]
            a = (ii - pbase) * 3
            px = plsc.load_gather(pts_v, [a])
            py = plsc.load_gather(pts_v, [a + 1])
            pz = plsc.load_gather(pts_v, [a + 2])
            oa = (g * _M + j * 16) * 3 + lane16 * 3
            plsc.store_scatter(out_v, [oa], px - cxv)
            plsc.store_scatter(out_v, [oa + 1], py - cyv)
            plsc.store_scatter(out_v, [oa + 2], pz - czv)
        return carry

    lax.fori_loop(0, gpt, group, 0)
    pltpu.sync_copy(out_v, out_hbm.at[pl.ds(gbase * _M * 3, gpt * _M * 3)])


def kernel(xyz):
    B, N, _ = xyz.shape
    xyzT = jnp.transpose(xyz, (0, 2, 1))

    cidx, cx, cy, cz = pl.pallas_call(
        _fps_body,
        out_shape=[
            jax.ShapeDtypeStruct((_B, _G), jnp.int32),
            jax.ShapeDtypeStruct((_B, _G), jnp.float32),
            jax.ShapeDtypeStruct((_B, _G), jnp.float32),
            jax.ShapeDtypeStruct((_B, _G), jnp.float32),
        ],
        in_specs=[pl.BlockSpec((_B, 3, _N), lambda: (0, 0, 0))],
        out_specs=[
            pl.BlockSpec((_B, _G), lambda: (0, 0)),
            pl.BlockSpec((_B, _G), lambda: (0, 0)),
            pl.BlockSpec((_B, _G), lambda: (0, 0)),
            pl.BlockSpec((_B, _G), lambda: (0, 0)),
        ],
        scratch_shapes=[pltpu.VMEM((_B, _N), jnp.float32)],
    )(xyzT)

    cx3 = cx.reshape(_B, 1, _G)
    cy3 = cy.reshape(_B, 1, _G)
    cz3 = cz.reshape(_B, 1, _G)
    idx = pl.pallas_call(
        _knn_body,
        grid=(_B, _G // _CB),
        out_shape=jax.ShapeDtypeStruct((_B, _G, _M), jnp.int32),
        in_specs=[
            pl.BlockSpec((1, 3, _N), lambda b, j: (b, 0, 0)),
            pl.BlockSpec((1, 1, _CB), lambda b, j: (b, 0, j)),
            pl.BlockSpec((1, 1, _CB), lambda b, j: (b, 0, j)),
            pl.BlockSpec((1, 1, _CB), lambda b, j: (b, 0, j)),
        ],
        out_specs=pl.BlockSpec((1, _CB, _M), lambda b, j: (b, j, 0)),
        scratch_shapes=[pltpu.VMEM((_CB, _N), jnp.float32)],
    )(xyzT, cx3, cy3, cz3)

    idx_flat = idx.reshape(-1)
    cidx_flat = cidx.reshape(-1)
    xyz_flat = xyz.reshape(-1)

    mesh = plsc.VectorSubcoreMesh(core_axis_name="c", subcore_axis_name="s")
    gpt = (_B * _G) // 32
    sc_gather = functools.partial(
        pl.kernel,
        mesh=mesh,
        out_type=jax.ShapeDtypeStruct((_B * _G * _M * 3,), jnp.float32),
        compiler_params=pltpu.CompilerParams(needs_layout_passes=False),
        scratch_types=[
            pltpu.VMEM((_N * 3,), jnp.float32),
            pltpu.VMEM((gpt * _M,), jnp.int32),
            pltpu.VMEM((gpt,), jnp.int32),
            pltpu.VMEM((gpt * _M * 3,), jnp.float32),
        ],
    )(_sc_gather_body)
    nb_flat = sc_gather(xyz_flat, idx_flat, cidx_flat)

    neighborhood = nb_flat.reshape(_B, _G, _M, 3)
    center = jnp.stack([cx, cy, cz], axis=-1)
    return neighborhood, center, idx_flat, cidx_flat


# speedup vs baseline: 1.2609x; 1.2609x over previous
import functools

import jax
import jax.numpy as jnp
from jax import lax
from jax.experimental import pallas as pl
from jax.experimental.pallas import tpu as pltpu
from jax.experimental.pallas import tpu_sc as plsc

_B = 8
_N = 8192
_G = 512
_M = 32
_CB = 256


def _fps_body(xT_ref, idx_ref, cx_ref, cy_ref, cz_ref, dist_ref):
    x = xT_ref[:, 0, :]
    y = xT_ref[:, 1, :]
    z = xT_ref[:, 2, :]
    lane = lax.broadcasted_iota(jnp.int32, (_B, _N), 1)
    boff = lax.broadcasted_iota(jnp.int32, (_B, 1), 0) * _N
    glane = lax.broadcasted_iota(jnp.int32, (_B, _G), 1)
    dist_ref[...] = jnp.full((_B, _N), 1e10, jnp.float32)

    def step(i, carry):
        idx_a, cx_a, cy_a, cz_a = carry
        d = dist_ref[...]
        m = jnp.max(d, axis=1, keepdims=True)
        f = jnp.min(jnp.where(d == m, lane, _N), axis=1, keepdims=True)
        sel = lane == f
        cx = jnp.sum(jnp.where(sel, x, 0.0), axis=1, keepdims=True)
        cy = jnp.sum(jnp.where(sel, y, 0.0), axis=1, keepdims=True)
        cz = jnp.sum(jnp.where(sel, z, 0.0), axis=1, keepdims=True)
        hot = glane == i
        idx_a = jnp.where(hot, f + boff, idx_a)
        cx_a = jnp.where(hot, cx, cx_a)
        cy_a = jnp.where(hot, cy, cy_a)
        cz_a = jnp.where(hot, cz, cz_a)
        dx = x - cx
        dy = y - cy
        dz = z - cz
        dist_ref[...] = jnp.minimum(d, dx * dx + dy * dy + dz * dz)
        return (idx_a, cx_a, cy_a, cz_a)

    init = (jnp.zeros((_B, _G), jnp.int32),
            jnp.zeros((_B, _G), jnp.float32),
            jnp.zeros((_B, _G), jnp.float32),
            jnp.zeros((_B, _G), jnp.float32))
    idx_a, cx_a, cy_a, cz_a = lax.fori_loop(0, _G, step, init)
    idx_ref[...] = idx_a
    cx_ref[...] = cx_a
    cy_ref[...] = cy_a
    cz_ref[...] = cz_a


def _knn_body(xT_ref, cx_ref, cy_ref, cz_ref, idx_ref, d_ref):
    b = pl.program_id(0)
    x = xT_ref[:, 0, :]
    y = xT_ref[:, 1, :]
    z = xT_ref[:, 2, :]
    rr = lax.broadcasted_iota(jnp.int32, (_CB, _CB), 0)
    cc = lax.broadcasted_iota(jnp.int32, (_CB, _CB), 1)
    eye = rr == cc

    def tocol(row_ref):
        row = jnp.broadcast_to(row_ref[...].reshape(1, _CB), (_CB, _CB))
        return jnp.sum(jnp.where(eye, row, 0.0), axis=1, keepdims=True)

    cxc = tocol(cx_ref)
    cyc = tocol(cy_ref)
    czc = tocol(cz_ref)
    dx = cxc - x
    dy = cyc - y
    dz = czc - z
    d_ref[...] = dx * dx + dy * dy + dz * dz
    lane = lax.broadcasted_iota(jnp.int32, (_CB, _N), 1)
    klane = lax.broadcasted_iota(jnp.int32, (_CB, _M), 1)
    off = b * _N

    bigf = jnp.float32(3.4e38)

    _E = 8

    def step(kk, idx_a):
        cur = d_ref[...]
        for j in range(_E):
            m = jnp.min(cur, axis=1, keepdims=True)
            a = jnp.min(jnp.where(cur == m, lane, _N), axis=1, keepdims=True)
            idx_a = jnp.where(klane == _E * kk + j, a + off, idx_a)
            cur = jnp.where(lane == a, bigf, cur)
        d_ref[...] = cur
        return idx_a

    idx_a = lax.fori_loop(0, _M // _E, step, jnp.zeros((_CB, _M), jnp.int32))
    idx_ref[0, :, :] = idx_a


def _sc_gather_body(xyz_hbm, idx_hbm, cidx_hbm, out_hbm,
                    pts_v, idx_v, cidx_v, out_v):
    gpt = (_B * _G) // 32
    wid = lax.axis_index("s") * 2 + lax.axis_index("c")
    gbase = wid * gpt
    b = gbase // _G
    pbase = b * _N
    pltpu.sync_copy(xyz_hbm.at[pl.ds(pbase * 3, _N * 3)], pts_v)
    pltpu.sync_copy(idx_hbm.at[pl.ds(gbase * _M, gpt * _M)], idx_v)
    pltpu.sync_copy(cidx_hbm.at[pl.ds(gbase, gpt)], cidx_v)
    lane16 = lax.broadcasted_iota(jnp.int32, (16,), 0)

    def group(g, carry):
        gg = jnp.full((16,), g, jnp.int32)
        ci = plsc.load_gather(cidx_v, [gg])
        ca = (ci - pbase) * 3
        cxv = plsc.load_gather(pts_v, [ca])
        cyv = plsc.load_gather(pts_v, [ca + 1])
        czv = plsc.load_gather(pts_v, [ca + 2])
        for j in range(_M // 16):
            ii = idx_v[pl.ds(g * _M + j * 16, 16)]
            a = (ii - pbase) * 3
            px = plsc.load_gather(pts_v, [a])
            py = plsc.load_gather(pts_v, [a + 1])
            pz = plsc.load_gather(pts_v, [a + 2])
            oa = (g * _M + j * 16) * 3 + lane16 * 3
            plsc.store_scatter(out_v, [oa], px - cxv)
            plsc.store_scatter(out_v, [oa + 1], py - cyv)
            plsc.store_scatter(out_v, [oa + 2], pz - czv)
        return carry

    lax.fori_loop(0, gpt, group, 0)
    pltpu.sync_copy(out_v, out_hbm.at[pl.ds(gbase * _M * 3, gpt * _M * 3)])


def kernel(xyz):
    B, N, _ = xyz.shape
    xyzT = jnp.transpose(xyz, (0, 2, 1))

    cidx, cx, cy, cz = pl.pallas_call(
        _fps_body,
        out_shape=[
            jax.ShapeDtypeStruct((_B, _G), jnp.int32),
            jax.ShapeDtypeStruct((_B, _G), jnp.float32),
            jax.ShapeDtypeStruct((_B, _G), jnp.float32),
            jax.ShapeDtypeStruct((_B, _G), jnp.float32),
        ],
        in_specs=[pl.BlockSpec((_B, 3, _N), lambda: (0, 0, 0))],
        out_specs=[
            pl.BlockSpec((_B, _G), lambda: (0, 0)),
            pl.BlockSpec((_B, _G), lambda: (0, 0)),
            pl.BlockSpec((_B, _G), lambda: (0, 0)),
            pl.BlockSpec((_B, _G), lambda: (0, 0)),
        ],
        scratch_shapes=[pltpu.VMEM((_B, _N), jnp.float32)],
    )(xyzT)

    cx3 = cx.reshape(_B, 1, _G)
    cy3 = cy.reshape(_B, 1, _G)
    cz3 = cz.reshape(_B, 1, _G)
    idx = pl.pallas_call(
        _knn_body,
        grid=(_B, _G // _CB),
        out_shape=jax.ShapeDtypeStruct((_B, _G, _M), jnp.int32),
        in_specs=[
            pl.BlockSpec((1, 3, _N), lambda b, j: (b, 0, 0)),
            pl.BlockSpec((1, 1, _CB), lambda b, j: (b, 0, j)),
            pl.BlockSpec((1, 1, _CB), lambda b, j: (b, 0, j)),
            pl.BlockSpec((1, 1, _CB), lambda b, j: (b, 0, j)),
        ],
        out_specs=pl.BlockSpec((1, _CB, _M), lambda b, j: (b, j, 0)),
        scratch_shapes=[pltpu.VMEM((_CB, _N), jnp.float32)],
    )(xyzT, cx3, cy3, cz3)

    idx_flat = idx.reshape(-1)
    cidx_flat = cidx.reshape(-1)
    xyz_flat = xyz.reshape(-1)

    mesh = plsc.VectorSubcoreMesh(core_axis_name="c", subcore_axis_name="s")
    gpt = (_B * _G) // 32
    sc_gather = functools.partial(
        pl.kernel,
        mesh=mesh,
        out_type=jax.ShapeDtypeStruct((_B * _G * _M * 3,), jnp.float32),
        compiler_params=pltpu.CompilerParams(needs_layout_passes=False),
        scratch_types=[
            pltpu.VMEM((_N * 3,), jnp.float32),
            pltpu.VMEM((gpt * _M,), jnp.int32),
            pltpu.VMEM((gpt,), jnp.int32),
            pltpu.VMEM((gpt * _M * 3,), jnp.float32),
        ],
    )(_sc_gather_body)
    nb_flat = sc_gather(xyz_flat, idx_flat, cidx_flat)

    neighborhood = nb_flat.reshape(_B, _G, _M, 3)
    center = jnp.stack([cx, cy, cz], axis=-1)
    return neighborhood, center, idx_flat, cidx_flat


# speedup vs baseline: 1.2615x; 1.0005x over previous
import functools

import jax
import jax.numpy as jnp
from jax import lax
from jax.experimental import pallas as pl
from jax.experimental.pallas import tpu as pltpu
from jax.experimental.pallas import tpu_sc as plsc

_B = 8
_N = 8192
_G = 512
_M = 32
_CB = 256


def _fps_body(xT_ref, idx_ref, cx_ref, cy_ref, cz_ref, dist_ref):
    x = xT_ref[0, :, :]
    y = xT_ref[1, :, :]
    z = xT_ref[2, :, :]
    p24 = xT_ref[...].reshape(3 * _B, _N)
    lane = lax.broadcasted_iota(jnp.int32, (_B, _N), 1)
    lane24 = lax.broadcasted_iota(jnp.int32, (3 * _B, _N), 1)
    boff = lax.broadcasted_iota(jnp.int32, (_B, 1), 0) * _N
    glane = lax.broadcasted_iota(jnp.int32, (_B, _G), 1)
    dist_ref[...] = jnp.full((_B, _N), 1e10, jnp.float32)

    def step(i, carry):
        idx_a, cx_a, cy_a, cz_a = carry
        d = dist_ref[...]
        m = jnp.max(d, axis=1, keepdims=True)
        f = jnp.min(jnp.where(d == m, lane, _N), axis=1, keepdims=True)
        f24 = jnp.concatenate([f, f, f], axis=0)
        csum = jnp.sum(jnp.where(lane24 == f24, p24, 0.0),
                       axis=1, keepdims=True)
        cx = csum[0:_B]
        cy = csum[_B:2 * _B]
        cz = csum[2 * _B:3 * _B]
        hot = glane == i
        idx_a = jnp.where(hot, f + boff, idx_a)
        cx_a = jnp.where(hot, cx, cx_a)
        cy_a = jnp.where(hot, cy, cy_a)
        cz_a = jnp.where(hot, cz, cz_a)
        dx = x - cx
        dy = y - cy
        dz = z - cz
        dist_ref[...] = jnp.minimum(d, dx * dx + dy * dy + dz * dz)
        return (idx_a, cx_a, cy_a, cz_a)

    init = (jnp.zeros((_B, _G), jnp.int32),
            jnp.zeros((_B, _G), jnp.float32),
            jnp.zeros((_B, _G), jnp.float32),
            jnp.zeros((_B, _G), jnp.float32))
    idx_a, cx_a, cy_a, cz_a = lax.fori_loop(0, _G, step, init)
    idx_ref[...] = idx_a
    cx_ref[...] = cx_a
    cy_ref[...] = cy_a
    cz_ref[...] = cz_a


def _knn_body(xT_ref, cx_ref, cy_ref, cz_ref, idx_ref, d_ref):
    b = pl.program_id(0)
    x = xT_ref[:, 0, :]
    y = xT_ref[:, 1, :]
    z = xT_ref[:, 2, :]
    rr = lax.broadcasted_iota(jnp.int32, (_CB, _CB), 0)
    cc = lax.broadcasted_iota(jnp.int32, (_CB, _CB), 1)
    eye = rr == cc

    def tocol(row_ref):
        row = jnp.broadcast_to(row_ref[...].reshape(1, _CB), (_CB, _CB))
        return jnp.sum(jnp.where(eye, row, 0.0), axis=1, keepdims=True)

    cxc = tocol(cx_ref)
    cyc = tocol(cy_ref)
    czc = tocol(cz_ref)
    dx = cxc - x
    dy = cyc - y
    dz = czc - z
    d_ref[...] = dx * dx + dy * dy + dz * dz
    lane = lax.broadcasted_iota(jnp.int32, (_CB, _N), 1)
    klane = lax.broadcasted_iota(jnp.int32, (_CB, _M), 1)
    off = b * _N

    bigf = jnp.float32(3.4e38)

    _E = 8

    def step(kk, idx_a):
        cur = d_ref[...]
        for j in range(_E):
            m = jnp.min(cur, axis=1, keepdims=True)
            a = jnp.min(jnp.where(cur == m, lane, _N), axis=1, keepdims=True)
            idx_a = jnp.where(klane == _E * kk + j, a + off, idx_a)
            cur = jnp.where(lane == a, bigf, cur)
        d_ref[...] = cur
        return idx_a

    idx_a = lax.fori_loop(0, _M // _E, step, jnp.zeros((_CB, _M), jnp.int32))
    idx_ref[0, :, :] = idx_a


def _sc_gather_body(xyz_hbm, idx_hbm, cidx_hbm, out_hbm,
                    pts_v, idx_v, cidx_v, out_v):
    gpt = (_B * _G) // 32
    wid = lax.axis_index("s") * 2 + lax.axis_index("c")
    gbase = wid * gpt
    b = gbase // _G
    pbase = b * _N
    pltpu.sync_copy(xyz_hbm.at[pl.ds(pbase * 3, _N * 3)], pts_v)
    pltpu.sync_copy(idx_hbm.at[pl.ds(gbase * _M, gpt * _M)], idx_v)
    pltpu.sync_copy(cidx_hbm.at[pl.ds(gbase, gpt)], cidx_v)
    lane16 = lax.broadcasted_iota(jnp.int32, (16,), 0)

    def group(g, carry):
        gg = jnp.full((16,), g, jnp.int32)
        ci = plsc.load_gather(cidx_v, [gg])
        ca = (ci - pbase) * 3
        cxv = plsc.load_gather(pts_v, [ca])
        cyv = plsc.load_gather(pts_v, [ca + 1])
        czv = plsc.load_gather(pts_v, [ca + 2])
        for j in range(_M // 16):
            ii = idx_v[pl.ds(g * _M + j * 16, 16)]
            a = (ii - pbase) * 3
            px = plsc.load_gather(pts_v, [a])
            py = plsc.load_gather(pts_v, [a + 1])
            pz = plsc.load_gather(pts_v, [a + 2])
            oa = (g * _M + j * 16) * 3 + lane16 * 3
            plsc.store_scatter(out_v, [oa], px - cxv)
            plsc.store_scatter(out_v, [oa + 1], py - cyv)
            plsc.store_scatter(out_v, [oa + 2], pz - czv)
        return carry

    lax.fori_loop(0, gpt, group, 0)
    pltpu.sync_copy(out_v, out_hbm.at[pl.ds(gbase * _M * 3, gpt * _M * 3)])


def kernel(xyz):
    B, N, _ = xyz.shape
    xyzC = jnp.transpose(xyz, (2, 0, 1))
    xyzT = jnp.transpose(xyz, (0, 2, 1))

    cidx, cx, cy, cz = pl.pallas_call(
        _fps_body,
        out_shape=[
            jax.ShapeDtypeStruct((_B, _G), jnp.int32),
            jax.ShapeDtypeStruct((_B, _G), jnp.float32),
            jax.ShapeDtypeStruct((_B, _G), jnp.float32),
            jax.ShapeDtypeStruct((_B, _G), jnp.float32),
        ],
        in_specs=[pl.BlockSpec((3, _B, _N), lambda: (0, 0, 0))],
        out_specs=[
            pl.BlockSpec((_B, _G), lambda: (0, 0)),
            pl.BlockSpec((_B, _G), lambda: (0, 0)),
            pl.BlockSpec((_B, _G), lambda: (0, 0)),
            pl.BlockSpec((_B, _G), lambda: (0, 0)),
        ],
        scratch_shapes=[pltpu.VMEM((_B, _N), jnp.float32)],
    )(xyzC)

    cx3 = cx.reshape(_B, 1, _G)
    cy3 = cy.reshape(_B, 1, _G)
    cz3 = cz.reshape(_B, 1, _G)
    idx = pl.pallas_call(
        _knn_body,
        grid=(_B, _G // _CB),
        out_shape=jax.ShapeDtypeStruct((_B, _G, _M), jnp.int32),
        in_specs=[
            pl.BlockSpec((1, 3, _N), lambda b, j: (b, 0, 0)),
            pl.BlockSpec((1, 1, _CB), lambda b, j: (b, 0, j)),
            pl.BlockSpec((1, 1, _CB), lambda b, j: (b, 0, j)),
            pl.BlockSpec((1, 1, _CB), lambda b, j: (b, 0, j)),
        ],
        out_specs=pl.BlockSpec((1, _CB, _M), lambda b, j: (b, j, 0)),
        scratch_shapes=[pltpu.VMEM((_CB, _N), jnp.float32)],
    )(xyzT, cx3, cy3, cz3)

    idx_flat = idx.reshape(-1)
    cidx_flat = cidx.reshape(-1)
    xyz_flat = xyz.reshape(-1)

    mesh = plsc.VectorSubcoreMesh(core_axis_name="c", subcore_axis_name="s")
    gpt = (_B * _G) // 32
    sc_gather = functools.partial(
        pl.kernel,
        mesh=mesh,
        out_type=jax.ShapeDtypeStruct((_B * _G * _M * 3,), jnp.float32),
        compiler_params=pltpu.CompilerParams(needs_layout_passes=False),
        scratch_types=[
            pltpu.VMEM((_N * 3,), jnp.float32),
            pltpu.VMEM((gpt * _M,), jnp.int32),
            pltpu.VMEM((gpt,), jnp.int32),
            pltpu.VMEM((gpt * _M * 3,), jnp.float32),
        ],
    )(_sc_gather_body)
    nb_flat = sc_gather(xyz_flat, idx_flat, cidx_flat)

    neighborhood = nb_flat.reshape(_B, _G, _M, 3)
    center = jnp.stack([cx, cy, cz], axis=-1)
    return neighborhood, center, idx_flat, cidx_flat
